# baseline (speedup 1.0000x reference)
import jax
import jax.numpy as jnp
from jax.experimental import pallas as pl
from jax.experimental.pallas import tpu as pltpu

_LANE = 128
_TH = 8


def _shift_cols_right(a):
    zero = jnp.zeros_like(a[:, :1, :])
    return jnp.concatenate([zero, a[:, :-1, :]], axis=1)


def _shift_cols_left(a):
    zero = jnp.zeros_like(a[:, :1, :])
    return jnp.concatenate([a[:, 1:, :], zero], axis=1)


def _rb_kernel(x_ref, w1_ref, w2_ref, b1_ref, b2_ref, out_ref,
               xlc_ref, xr2_ref, hlc_ref, hr2_ref, w1s_ref, w2s_ref):
    H, W, C = x_ref.shape
    MS = _TH * W
    NS = H // _TH

    def _pack():
        zc = jnp.zeros((C, C), jnp.bfloat16)
        for ws, w in ((w1s_ref, w1_ref), (w2s_ref, w2_ref)):
            ws[0, 0:C, 0:C] = w[0, 0]
            ws[0, 0:C, C:2 * C] = w[1, 0]
            ws[0, C:2 * C, 0:C] = w[0, 1]
            ws[0, C:2 * C, C:2 * C] = w[1, 1]
            ws[1, 0:C, 0:C] = zc
            ws[1, 0:C, C:2 * C] = w[2, 0]
            ws[1, C:2 * C, 0:C] = zc
            ws[1, C:2 * C, C:2 * C] = w[2, 1]
            ws[2, 0:C, 0:C] = w[0, 2]
            ws[2, 0:C, C:2 * C] = w[1, 2]
            ws[2, C:2 * C, 0:C] = zc
            ws[2, C:2 * C, C:2 * C] = w[2, 2]
    _pack()

    zrow = jnp.zeros((1, W, 2 * C), jnp.bfloat16)
    for ref in (xlc_ref, xr2_ref, hlc_ref, hr2_ref):
        ref[0:1] = zrow
        ref[H + 1:H + 2] = zrow
        ref[H + 2:H + 3] = zrow
    xr2_ref[H:H + 1, :, C:2 * C] = zrow[:, :, 0:C]
    hr2_ref[H:H + 1, :, C:2 * C] = zrow[:, :, 0:C]

    def _fill(lc_ref, r2_ref, v):
        lc_ref[1:H + 1, :, 0:C] = _shift_cols_right(v)
        lc_ref[1:H + 1, :, C:2 * C] = v
        vr = _shift_cols_left(v)
        r2_ref[1:H + 1, :, 0:C] = vr
        r2_ref[0:H, :, C:2 * C] = vr

    def _pop_strip(lc_ref, r2_ref, ws_ref, region, s):
        r0 = _TH * s
        nrows = 2 if s == NS else _TH
        mxu = s % 2
        for k, sr in ((0, 0), (1, 1), (2, 0)):
            if k == 0:
                blk = lc_ref[r0:r0 + nrows]
            elif k == 1:
                blk = lc_ref[r0 + 1:r0 + 1 + nrows]
            else:
                blk = r2_ref[r0:r0 + nrows]
            pltpu.matmul_push_rhs(ws_ref[k], staging_register=sr,
                                  mxu_index=mxu)
            pltpu.matmul_acc_lhs(
                acc_addr=region, lhs=blk.reshape(nrows * W, 2 * C),
                mxu_index=mxu, load_staged_rhs=sr)
        return pltpu.matmul_pop(acc_addr=region,
                                shape=(nrows * W, 2 * C),
                                dtype=jnp.float32, mxu_index=mxu)

    def _combine(s, r, rn, extra_fn, emit):
        blk1 = jnp.concatenate(
            [r[W:MS, C:2 * C], rn[0:W, C:2 * C]], axis=0)
        emit(s, r[:, 0:C] + blk1 + extra_fn(s))

    _fill(xlc_ref, xr2_ref, x_ref[...].astype(jnp.bfloat16))

    def extra1(s):
        return b1_ref[...]

    def extra2(s):
        r0 = s * _TH
        return (b2_ref[...]
                + x_ref[r0:r0 + _TH].astype(jnp.float32).reshape(MS, C))

    def emit1(s, y):
        h = jnp.maximum(y, 0.0).reshape(_TH, W, C).astype(jnp.bfloat16)
        r0 = s * _TH
        hlc_ref[1 + r0:1 + r0 + _TH, :, 0:C] = _shift_cols_right(h)
        hlc_ref[1 + r0:1 + r0 + _TH, :, C:2 * C] = h
        hr = _shift_cols_left(h)
        hr2_ref[1 + r0:1 + r0 + _TH, :, 0:C] = hr
        hr2_ref[r0:r0 + _TH, :, C:2 * C] = hr

    def emit2(s, y):
        r0 = s * _TH
        out_ref[r0:r0 + _TH] = jnp.maximum(y, 0.0).reshape(
            _TH, W, C).astype(out_ref.dtype)

    p1 = {}
    p2 = {}

    def c1(s):
        p1[s] = _pop_strip(xlc_ref, xr2_ref, w1s_ref, 0, s)
        if s > 0:
            _combine(s - 1, p1[s - 1], p1[s], extra1, emit1)

    def c2(s):
        p2[s] = _pop_strip(hlc_ref, hr2_ref, w2s_ref, 128, s)
        if s > 0:
            _combine(s - 1, p2[s - 1], p2[s], extra2, emit2)

    c1(0)
    c1(1)
    c1(2)
    for s in range(NS - 2):
        c2(s)
        c1(s + 3)
    c2(NS - 2)
    c2(NS - 1)
    c2(NS)


def kernel(x_nhwc, w1f, bias1, w2f, bias2):
    N, H, W, C = x_nhwc.shape
    assert C % _LANE == 0 and W % 8 == 0 and H % _TH == 0 and W >= 16, \
        (N, H, W, C)

    b1 = bias1.astype(jnp.float32).reshape(1, C)
    b2 = bias2.astype(jnp.float32).reshape(1, C)

    def const_spec(shape):
        return pl.BlockSpec(shape, lambda n: tuple(0 for _ in shape),
                            pipeline_mode=pl.Buffered(1))

    return pl.pallas_call(
        _rb_kernel,
        out_shape=jax.ShapeDtypeStruct((N, H, W, C), x_nhwc.dtype),
        grid=(N,),
        in_specs=[
            pl.BlockSpec((None, H, W, C), lambda n: (n, 0, 0, 0)),
            const_spec((3, 3, C, C)),
            const_spec((3, 3, C, C)),
            const_spec((1, C)),
            const_spec((1, C)),
        ],
        out_specs=pl.BlockSpec((None, H, W, C), lambda n: (n, 0, 0, 0)),
        scratch_shapes=(
            [pltpu.VMEM((H + 3, W, 2 * C), jnp.bfloat16)
             for _ in range(4)]
            + [pltpu.VMEM((3, 2 * C, 2 * C), jnp.bfloat16)
               for _ in range(2)]),
        compiler_params=pltpu.CompilerParams(
            dimension_semantics=("parallel",)),
    )(x_nhwc, w1f.astype(jnp.bfloat16), w2f.astype(jnp.bfloat16), b1, b2)


# speedup vs baseline: 1.0524x; 1.0524x over previous
import jax
import jax.numpy as jnp
from jax.experimental import pallas as pl
from jax.experimental.pallas import tpu as pltpu

_LANE = 128
_TH = 7
_IMGS = 2


def _shift_cols_right(a):
    zero = jnp.zeros_like(a[:, :1, :])
    return jnp.concatenate([zero, a[:, :-1, :]], axis=1)


def _shift_cols_left(a):
    zero = jnp.zeros_like(a[:, :1, :])
    return jnp.concatenate([a[:, 1:, :], zero], axis=1)


def _rb_kernel(x_ref, w1_ref, w2_ref, b1_ref, b2_ref, out_ref,
               xlc_ref, xr2_ref, hlc_ref, hr2_ref, w1s_ref, w2s_ref):
    _, H, W, C = x_ref.shape
    MS = _TH * W
    MD = MS + W
    NS = H // _TH

    def _pack():
        zc = jnp.zeros((C, C), jnp.bfloat16)
        for ws, w in ((w1s_ref, w1_ref), (w2s_ref, w2_ref)):
            ws[0, 0:C, 0:C] = w[0, 0]
            ws[0, 0:C, C:2 * C] = w[1, 0]
            ws[0, C:2 * C, 0:C] = w[0, 1]
            ws[0, C:2 * C, C:2 * C] = w[1, 1]
            ws[1, 0:C, 0:C] = zc
            ws[1, 0:C, C:2 * C] = w[2, 0]
            ws[1, C:2 * C, 0:C] = zc
            ws[1, C:2 * C, C:2 * C] = w[2, 1]
            ws[2, 0:C, 0:C] = w[0, 2]
            ws[2, 0:C, C:2 * C] = w[1, 2]
            ws[2, C:2 * C, 0:C] = zc
            ws[2, C:2 * C, C:2 * C] = w[2, 2]
    _pack()

    zrow = jnp.zeros((1, W, 2 * C), jnp.bfloat16)
    for img in range(_IMGS):
        for ref in (xlc_ref, xr2_ref, hlc_ref, hr2_ref):
            ref[img, 0:1] = zrow
            ref[img, H + 1:H + 2] = zrow
        xr2_ref[img, H:H + 1, :, C:2 * C] = zrow[:, :, 0:C]
        hr2_ref[img, H:H + 1, :, C:2 * C] = zrow[:, :, 0:C]

    def _fill(lc_ref, r2_ref, img, v):
        lc_ref[img, 1:H + 1, :, 0:C] = _shift_cols_right(v)
        lc_ref[img, 1:H + 1, :, C:2 * C] = v
        vr = _shift_cols_left(v)
        r2_ref[img, 1:H + 1, :, 0:C] = vr
        r2_ref[img, 0:H, :, C:2 * C] = vr

    def _strip(lc_ref, r2_ref, img, ws_ref, region, s, extra_fn, emit):
        mxu = s % 2
        for k, sr in ((0, 0), (1, 1), (2, 0)):
            r0 = _TH * s
            if k == 0:
                blk = lc_ref[img, r0:r0 + _TH + 1]
            elif k == 1:
                blk = lc_ref[img, r0 + 1:r0 + _TH + 2]
            else:
                blk = r2_ref[img, r0:r0 + _TH + 1]
            pltpu.matmul_push_rhs(ws_ref[k], staging_register=sr,
                                  mxu_index=mxu)
            pltpu.matmul_acc_lhs(
                acc_addr=region, lhs=blk.reshape(MD, 2 * C),
                mxu_index=mxu, load_staged_rhs=sr)
        r = pltpu.matmul_pop(acc_addr=region, shape=(MD, 2 * C),
                             dtype=jnp.float32, mxu_index=mxu)
        y = r[0:MS, 0:C] + r[W:MD, C:2 * C] + extra_fn(s)
        emit(s, y)

    def extra1(s):
        return b1_ref[...]

    def extra2(img):
        def go(s):
            r0 = s * _TH
            return (b2_ref[...] + x_ref[img, r0:r0 + _TH]
                    .astype(jnp.float32).reshape(MS, C))
        return go

    def emit1(img):
        def go(s, y):
            h = jnp.maximum(y, 0.0).reshape(_TH, W, C).astype(jnp.bfloat16)
            r0 = s * _TH
            hlc_ref[img, 1 + r0:1 + r0 + _TH, :, 0:C] = \
                _shift_cols_right(h)
            hlc_ref[img, 1 + r0:1 + r0 + _TH, :, C:2 * C] = h
            hr = _shift_cols_left(h)
            hr2_ref[img, 1 + r0:1 + r0 + _TH, :, 0:C] = hr
            hr2_ref[img, r0:r0 + _TH, :, C:2 * C] = hr
        return go

    def emit2(img):
        def go(s, y):
            r0 = s * _TH
            out_ref[img, r0:r0 + _TH] = jnp.maximum(y, 0.0).reshape(
                _TH, W, C).astype(out_ref.dtype)
        return go

    def c1(img, s):
        _strip(xlc_ref, xr2_ref, img, w1s_ref, 0, s,
               extra1, emit1(img))

    def c2(img, s):
        _strip(hlc_ref, hr2_ref, img, w2s_ref, MD // 4, s,
               extra2(img), emit2(img))

    _fill(xlc_ref, xr2_ref, 0, x_ref[0].astype(jnp.bfloat16))
    c1(0, 0)
    c1(0, 1)
    _fill(xlc_ref, xr2_ref, 1, x_ref[1].astype(jnp.bfloat16))
    for s in range(NS - 2):
        c1(0, s + 2)
        c2(0, s)
    c1(1, 0)
    c2(0, NS - 2)
    c1(1, 1)
    c2(0, NS - 1)
    for s in range(NS - 2):
        c1(1, s + 2)
        c2(1, s)
    c2(1, NS - 2)
    c2(1, NS - 1)


def kernel(x_nhwc, w1f, bias1, w2f, bias2):
    N, H, W, C = x_nhwc.shape
    assert (C % _LANE == 0 and W % 8 == 0 and H % (4 * _TH) == 0
            and N % _IMGS == 0), (N, H, W, C)

    b1 = bias1.astype(jnp.float32).reshape(1, C)
    b2 = bias2.astype(jnp.float32).reshape(1, C)

    def const_spec(shape):
        return pl.BlockSpec(shape, lambda n: tuple(0 for _ in shape),
                            pipeline_mode=pl.Buffered(1))

    return pl.pallas_call(
        _rb_kernel,
        out_shape=jax.ShapeDtypeStruct((N, H, W, C), x_nhwc.dtype),
        grid=(N // _IMGS,),
        in_specs=[
            pl.BlockSpec((_IMGS, H, W, C), lambda n: (n, 0, 0, 0)),
            const_spec((3, 3, C, C)),
            const_spec((3, 3, C, C)),
            const_spec((1, C)),
            const_spec((1, C)),
        ],
        out_specs=pl.BlockSpec((_IMGS, H, W, C), lambda n: (n, 0, 0, 0)),
        scratch_shapes=(
            [pltpu.VMEM((_IMGS, H + 2, W, 2 * C), jnp.bfloat16)
             for _ in range(4)]
            + [pltpu.VMEM((3, 2 * C, 2 * C), jnp.bfloat16)
               for _ in range(2)]),
        compiler_params=pltpu.CompilerParams(
            dimension_semantics=("parallel",)),
    )(x_nhwc, w1f.astype(jnp.bfloat16), w2f.astype(jnp.bfloat16), b1, b2)
